# baseline (speedup 1.0000x reference)
import functools

import jax
import jax.numpy as jnp
from jax import lax
from jax.experimental import pallas as pl
from jax.experimental.pallas import tpu as pltpu
from jax.experimental.pallas import tpu_sc as plsc

EMB = 64
NC = 2
NS = 16
NW = NC * NS

CHUNK = 128


def _sc_gather(ids, table, n_rows):
    b_per_w = n_rows // NW
    n_chunks = b_per_w // CHUNK
    mesh = plsc.VectorSubcoreMesh(core_axis_name="c", subcore_axis_name="s")

    @functools.partial(
        pl.kernel,
        out_type=jax.ShapeDtypeStruct((n_rows, EMB), jnp.float32),
        mesh=mesh,
        scratch_types=[
            pltpu.VMEM((b_per_w,), jnp.int32),
            pltpu.VMEM((CHUNK, EMB), jnp.float32),
            pltpu.SemaphoreType.DMA,
        ],
        compiler_params=pltpu.CompilerParams(use_tc_tiling_on_sc=False),
    )
    def k(ids_hbm, table_hbm, out_hbm, idx_v, buf, sem):
        wid = lax.axis_index("s") * NC + lax.axis_index("c")
        base = wid * b_per_w
        pltpu.sync_copy(ids_hbm.at[pl.ds(base, b_per_w)], idx_v)

        def body(g, carry):
            off = g * CHUNK
            pltpu.async_copy(
                table_hbm.at[idx_v.at[pl.ds(off, CHUNK)]], buf, sem
            ).wait()
            pltpu.sync_copy(buf, out_hbm.at[pl.ds(base + off, CHUNK)])
            return carry

        lax.fori_loop(0, n_chunks, body, 0)

    return k(ids, table)


def _tc_dense(x, pos, W, b, gamma, beta):
    Bn, S, E = x.shape
    BB = 16
    R = BB * S

    def body(x_ref, pos_ref, w_ref, b_ref, g_ref, be_ref, o_ref):
        xp = x_ref[...] + pos_ref[...][None]
        y = lax.dot_general(
            xp.reshape(R, E), w_ref[...],
            (((1,), (1,)), ((), ())),
            preferred_element_type=jnp.float32,
            precision=lax.Precision.HIGHEST,
        )
        y = y + b_ref[...]
        mu = jnp.mean(y, axis=1, keepdims=True)
        d = y - mu
        var = jnp.mean(d * d, axis=1, keepdims=True)
        o = d * lax.rsqrt(var + 1e-5) * g_ref[...] + be_ref[...]
        o_ref[...] = o.reshape(BB, S, E)

    return pl.pallas_call(
        body,
        grid=(Bn // BB,),
        in_specs=[
            pl.BlockSpec((BB, S, E), lambda i: (i, 0, 0)),
            pl.BlockSpec((S, E), lambda i: (0, 0)),
            pl.BlockSpec((E, E), lambda i: (0, 0)),
            pl.BlockSpec((1, E), lambda i: (0, 0)),
            pl.BlockSpec((1, E), lambda i: (0, 0)),
            pl.BlockSpec((1, E), lambda i: (0, 0)),
        ],
        out_specs=pl.BlockSpec((BB, S, E), lambda i: (i, 0, 0)),
        out_shape=jax.ShapeDtypeStruct((Bn, S, E), jnp.float32),
    )(x, pos, W, b.reshape(1, E), gamma.reshape(1, E), beta.reshape(1, E))


def kernel(input_ids, tok_table, pos_table, W, b, gamma, beta):
    Bn, S = input_ids.shape
    ids = input_ids.reshape(-1).astype(jnp.int32)
    tok = _sc_gather(ids, tok_table, Bn * S)
    x = tok.reshape(Bn, S, EMB)
    return _tc_dense(x, pos_table, W, b, gamma, beta)


# speedup vs baseline: 1.1795x; 1.1795x over previous
import functools

import jax
import jax.numpy as jnp
from jax import lax
from jax.experimental import pallas as pl
from jax.experimental.pallas import tpu as pltpu
from jax.experimental.pallas import tpu_sc as plsc

EMB = 64
NC = 2
NS = 16
NW = NC * NS

CHUNK = 128


def _sc_gather(ids, table, n_rows):
    b_per_w = n_rows // NW
    n_chunks = b_per_w // CHUNK
    mesh = plsc.VectorSubcoreMesh(core_axis_name="c", subcore_axis_name="s")

    @functools.partial(
        pl.kernel,
        out_type=jax.ShapeDtypeStruct((n_rows, EMB), jnp.float32),
        mesh=mesh,
        scratch_types=[
            pltpu.VMEM((b_per_w,), jnp.int32),
            pltpu.VMEM((CHUNK, EMB), jnp.float32),
            pltpu.SemaphoreType.DMA,
        ],
        compiler_params=pltpu.CompilerParams(use_tc_tiling_on_sc=False),
    )
    def k(ids_hbm, table_hbm, out_hbm, idx_v, buf, sem):
        wid = lax.axis_index("s") * NC + lax.axis_index("c")
        base = wid * b_per_w
        out_flat = out_hbm
        pltpu.sync_copy(ids_hbm.at[pl.ds(base, b_per_w)], idx_v)

        def body(g, carry):
            off = g * CHUNK
            pltpu.async_copy(
                table_hbm.at[idx_v.at[pl.ds(off, CHUNK)]], buf, sem
            ).wait()
            pltpu.sync_copy(buf, out_flat.at[pl.ds(base + off, CHUNK)])
            return carry

        lax.fori_loop(0, n_chunks, body, 0)

    return k(ids, table)


def _tc_dense(x2, pos2, W2, b2, g2, be2):
    N2, L = x2.shape
    SP = pos2.shape[0]
    RB = 3200
    reps = RB // SP

    H = jnp.kron(jnp.eye(2, dtype=jnp.float32),
                 jnp.full((EMB, EMB), 1.0 / EMB, dtype=jnp.float32))

    def body(x_ref, pos_ref, w_ref, h_ref, b_ref, g_ref, be_ref, o_ref):
        x = x_ref[...]
        xp = (x.reshape(reps, SP, L) + pos_ref[...][None]).reshape(RB, L)
        y = lax.dot_general(
            xp, w_ref[...], (((1,), (0,)), ((), ())),
            preferred_element_type=jnp.float32,
            precision=lax.Precision.HIGHEST,
        ) + b_ref[...]
        mu = lax.dot_general(
            y, h_ref[...], (((1,), (0,)), ((), ())),
            preferred_element_type=jnp.float32,
            precision=lax.Precision.DEFAULT,
        )
        ysq = lax.dot_general(
            y * y, h_ref[...], (((1,), (0,)), ((), ())),
            preferred_element_type=jnp.float32,
            precision=lax.Precision.DEFAULT,
        )
        var = ysq - mu * mu
        o_ref[...] = (y - mu) * lax.rsqrt(var + 1e-5) * g_ref[...] + be_ref[...]

    return pl.pallas_call(
        body,
        grid=(N2 // RB,),
        in_specs=[
            pl.BlockSpec((RB, L), lambda i: (i, 0)),
            pl.BlockSpec((SP, L), lambda i: (0, 0)),
            pl.BlockSpec((L, L), lambda i: (0, 0)),
            pl.BlockSpec((L, L), lambda i: (0, 0)),
            pl.BlockSpec((1, L), lambda i: (0, 0)),
            pl.BlockSpec((1, L), lambda i: (0, 0)),
            pl.BlockSpec((1, L), lambda i: (0, 0)),
        ],
        out_specs=pl.BlockSpec((RB, L), lambda i: (i, 0)),
        out_shape=jax.ShapeDtypeStruct((N2, L), jnp.float32),
    )(x2, pos2, W2, H, b2, g2, be2)


def kernel(input_ids, tok_table, pos_table, W, b, gamma, beta):
    Bn, S = input_ids.shape
    n_rows = Bn * S
    ids = input_ids.reshape(-1).astype(jnp.int32)
    x2 = _sc_gather(ids, tok_table, n_rows).reshape(n_rows // 2, 2 * EMB)

    Wt = W.T
    Z = jnp.zeros((EMB, EMB), dtype=jnp.float32)
    W2 = jnp.block([[Wt, Z], [Z, Wt]])
    pos2 = pos_table.reshape(S // 2, 2 * EMB)
    b2 = jnp.concatenate([b, b]).reshape(1, 2 * EMB)
    g2 = jnp.concatenate([gamma, gamma]).reshape(1, 2 * EMB)
    be2 = jnp.concatenate([beta, beta]).reshape(1, 2 * EMB)

    out2 = _tc_dense(x2, pos2, W2, b2, g2, be2)
    return out2.reshape(Bn, S, EMB)
